# baseline (speedup 1.0000x reference)
import functools

import jax
import jax.numpy as jnp
from jax import lax
from jax.experimental import pallas as pl
from jax.experimental.pallas import tpu as pltpu
from jax.experimental.pallas import tpu_sc as plsc

N = 10000
D = 128
NQ = 4
DQ = D // NQ
G = 64
E = 320000

NPAD = 10240
R = 512
NGRID = NPAD // R
NC = 2
NS = 16
NW = NC * NS
B = 512
CH = 20
EPAD = NW * CH * B
RPT = NPAD // NS
ZR = 128
NBUF = 4
HR = 32
HSEG = NPAD // HR

_sc_params = pltpu.CompilerParams(needs_layout_passes=False,
                                  use_tc_tiling_on_sc=False)


@functools.lru_cache(maxsize=None)
def _sc_kernels():
    mesh = plsc.VectorSubcoreMesh(core_axis_name="c", subcore_axis_name="s",
                                  num_cores=NC, num_subcores=NS)
    deg = functools.partial(
        pl.kernel,
        out_type=jax.ShapeDtypeStruct((NC, HR, HSEG), jnp.float32),
        mesh=mesh,
        compiler_params=_sc_params,
        scratch_types=[
            pltpu.VMEM((CH, B), jnp.int32),
            pltpu.VMEM((HR, HSEG), jnp.float32),
            pltpu.VMEM((HR,), jnp.int32),
            pltpu.VMEM_SHARED((HR, HSEG), jnp.float32),
        ],
    )(_deg_body)
    edge = functools.partial(
        pl.kernel,
        out_type=jax.ShapeDtypeStruct((NC, NQ, NPAD, DQ), jnp.float32),
        mesh=mesh,
        compiler_params=_sc_params,
        scratch_types=[
            pltpu.VMEM((CH, B), jnp.int32),
            pltpu.VMEM((CH, B), jnp.int32),
            [pltpu.VMEM((B, DQ), jnp.float32) for _ in range(NBUF)],
            pltpu.VMEM((ZR, DQ), jnp.float32),
            pltpu.VMEM_SHARED((NPAD, DQ), jnp.float32),
            pltpu.VMEM_SHARED((NPAD, DQ), jnp.float32),
            [pltpu.SemaphoreType.DMA for _ in range(NBUF)],
            [pltpu.SemaphoreType.DMA for _ in range(NBUF)],
        ],
    )(_edge_body)
    return deg, edge


def _deg_body(dst_hbm, out_hbm, idx_v, hist_v, iref, acc_sh):
    c = lax.axis_index("c")
    s = lax.axis_index("s")
    w = s * NC + c
    pltpu.sync_copy(dst_hbm.at[w], idx_v)
    zero16 = jnp.zeros((16,), jnp.float32)

    def _zero(i, carry):
        for k in range(HSEG // 16):
            hist_v[i, pl.ds(k * 16, 16)] = zero16
        return carry

    lax.fori_loop(0, HR, _zero, 0)

    @pl.when(s == 0)
    def _():
        pltpu.sync_copy(hist_v, acc_sh)

    plsc.subcore_barrier()

    one16 = jnp.ones((16,), jnp.float32)

    def _hist(j, carry):
        for k in range(B // 16):
            idx = idx_v[j, pl.ds(k * 16, 16)]
            plsc.addupdate_scatter(hist_v, [idx // HSEG, idx % HSEG], one16)
        return carry

    lax.fori_loop(0, CH, _hist, 0)

    iota16 = lax.iota(jnp.int32, 16)
    iref[pl.ds(0, 16)] = iota16
    iref[pl.ds(16, 16)] = iota16 + 16
    pltpu.sync_copy(hist_v, acc_sh.at[iref], add=True)
    plsc.subcore_barrier()
    pltpu.sync_copy(acc_sh.at[pl.ds(2 * s, 2)], out_hbm.at[c, pl.ds(2 * s, 2)])


def _edge_body(hq0_hbm, hq1_hbm, hq2_hbm, hq3_hbm, src_hbm, dst_hbm, out_hbm,
               sidx, didx, bufs, zbuf, hsh, acc, gsems, ssems):
    c = lax.axis_index("c")
    s = lax.axis_index("s")
    w = s * NC + c
    pltpu.sync_copy(src_hbm.at[w], sidx)
    pltpu.sync_copy(dst_hbm.at[w], didx)

    zero16 = jnp.zeros((16,), jnp.float32)

    def _zb(i, carry):
        for k in range(DQ // 16):
            zbuf[i, pl.ds(k * 16, 16)] = zero16
        return carry

    lax.fori_loop(0, ZR, _zb, 0)

    base = s * RPT

    for p, hq_hbm in enumerate((hq0_hbm, hq1_hbm, hq2_hbm, hq3_hbm)):
        pltpu.sync_copy(hq_hbm.at[pl.ds(base, RPT)], hsh.at[pl.ds(base, RPT)])

        def _za(i, carry):
            pltpu.sync_copy(zbuf, acc.at[pl.ds(base + i * ZR, ZR)])
            return carry

        lax.fori_loop(0, RPT // ZR, _za, 0)
        plsc.subcore_barrier()

        pltpu.async_copy(hsh.at[sidx.at[0]], bufs[0], gsems[0])
        pltpu.async_copy(hsh.at[sidx.at[1]], bufs[1], gsems[1])

        def _body(k, carry):
            v0 = NBUF * k
            for b in range(NBUF):
                v = v0 + b
                pltpu.make_async_copy(hsh.at[sidx.at[v]], bufs[b],
                                      gsems[b]).wait()
                pltpu.async_copy(bufs[b], acc.at[didx.at[v]], ssems[b],
                                 add=True)
                bg = (b + 2) % NBUF

                @pl.when(v >= 2)
                def _():
                    pltpu.make_async_copy(bufs[bg], acc.at[didx.at[v]],
                                          ssems[bg]).wait()

                @pl.when(v + 2 < CH)
                def _():
                    pltpu.async_copy(hsh.at[sidx.at[v + 2]], bufs[bg],
                                     gsems[bg])

            return carry

        lax.fori_loop(0, CH // NBUF, _body, 0)
        for j in (CH - 2, CH - 1):
            pltpu.make_async_copy(bufs[j % NBUF], acc.at[didx.at[j]],
                                  ssems[j % NBUF]).wait()
        plsc.subcore_barrier()
        pltpu.sync_copy(acc.at[pl.ds(base, RPT)],
                        out_hbm.at[c, p, pl.ds(base, RPT)])


def _mm(a, b):
    return lax.dot_general(a, b, (((1,), (0,)), ((), ())),
                           preferred_element_type=jnp.float32,
                           precision=lax.Precision.HIGHEST)


def _tc1_body(x_ref, w_ref, degp_ref, hq0_ref, hq1_ref, hq2_ref, hq3_ref,
              dinv_ref):
    deg = degp_ref[0] + degp_ref[1] + 1.0
    dinv = lax.rsqrt(deg)
    h = dinv * _mm(x_ref[...], w_ref[...])
    for q, ref in enumerate((hq0_ref, hq1_ref, hq2_ref, hq3_ref)):
        ref[...] = h[:, q * DQ:(q + 1) * DQ]
    dinv_ref[...] = dinv


_tc1 = pl.pallas_call(
    _tc1_body,
    grid=(NGRID,),
    in_specs=[
        pl.BlockSpec((R, D), lambda i: (i, 0)),
        pl.BlockSpec((D, D), lambda i: (0, 0)),
        pl.BlockSpec((NC, R, 1), lambda i: (0, i, 0)),
    ],
    out_specs=[pl.BlockSpec((R, DQ), lambda i: (i, 0)) for _ in range(NQ)]
    + [pl.BlockSpec((R, 1), lambda i: (i, 0))],
    out_shape=[jax.ShapeDtypeStruct((NPAD, DQ), jnp.float32)
               for _ in range(NQ)]
    + [jax.ShapeDtypeStruct((NPAD, 1), jnp.float32)],
)


def _node_emb(s_ref, hq_refs, dinv_ref, b_ref):
    dinv = dinv_ref[...]
    quarters = [dinv * (s_ref[0, q] + s_ref[1, q] + hq_refs[q][...])
                for q in range(NQ)]
    return jnp.concatenate(quarters, axis=1) + b_ref[...]


def _tc2_body(s_ref, hq0_ref, hq1_ref, hq2_ref, hq3_ref, dinv_ref, b1_ref,
              w2_ref, oq0_ref, oq1_ref, oq2_ref, oq3_ref):
    h1 = jnp.maximum(
        _node_emb(s_ref, (hq0_ref, hq1_ref, hq2_ref, hq3_ref), dinv_ref,
                  b1_ref), 0.0)
    h2 = dinv_ref[...] * _mm(h1, w2_ref[...])
    for q, ref in enumerate((oq0_ref, oq1_ref, oq2_ref, oq3_ref)):
        ref[...] = h2[:, q * DQ:(q + 1) * DQ]


_tc2 = pl.pallas_call(
    _tc2_body,
    grid=(NGRID,),
    in_specs=[
        pl.BlockSpec((NC, NQ, R, DQ), lambda i: (0, 0, i, 0)),
    ] + [pl.BlockSpec((R, DQ), lambda i: (i, 0)) for _ in range(NQ)] + [
        pl.BlockSpec((R, 1), lambda i: (i, 0)),
        pl.BlockSpec((1, D), lambda i: (0, 0)),
        pl.BlockSpec((D, D), lambda i: (0, 0)),
    ],
    out_specs=[pl.BlockSpec((R, DQ), lambda i: (i, 0)) for _ in range(NQ)],
    out_shape=[jax.ShapeDtypeStruct((NPAD, DQ), jnp.float32)
               for _ in range(NQ)],
)


def _tc3_body(s_ref, hq0_ref, hq1_ref, hq2_ref, hq3_ref, dinv_ref, b2_ref,
              batch_ref, out_ref, acc_ref, cnt_ref):
    i = pl.program_id(0)
    node = _node_emb(s_ref, (hq0_ref, hq1_ref, hq2_ref, hq3_ref), dinv_ref,
                     b2_ref)
    batch_blk = batch_ref[0]
    iota_g = lax.broadcasted_iota(jnp.int32, (G, R), 0)
    mask = (batch_blk == iota_g).astype(jnp.float32)
    contrib = _mm(mask, node)
    cntc = jnp.sum(mask, axis=1, keepdims=True)

    @pl.when(i == 0)
    def _():
        acc_ref[...] = contrib
        cnt_ref[...] = cntc

    @pl.when(i > 0)
    def _():
        acc_ref[...] += contrib
        cnt_ref[...] += cntc

    @pl.when(i == NGRID - 1)
    def _():
        out_ref[...] = acc_ref[...] / jnp.maximum(cnt_ref[...], 1.0)


_tc3 = pl.pallas_call(
    _tc3_body,
    grid=(NGRID,),
    in_specs=[
        pl.BlockSpec((NC, NQ, R, DQ), lambda i: (0, 0, i, 0)),
    ] + [pl.BlockSpec((R, DQ), lambda i: (i, 0)) for _ in range(NQ)] + [
        pl.BlockSpec((R, 1), lambda i: (i, 0)),
        pl.BlockSpec((1, D), lambda i: (0, 0)),
        pl.BlockSpec((1, 1, R), lambda i: (i, 0, 0)),
    ],
    out_specs=pl.BlockSpec((G, D), lambda i: (0, 0)),
    out_shape=jax.ShapeDtypeStruct((G, D), jnp.float32),
    scratch_shapes=[
        pltpu.VMEM((G, D), jnp.float32),
        pltpu.VMEM((G, 1), jnp.float32),
    ],
)


def kernel(x, edge_index, batch, W1, b1, W2, b2):
    src = edge_index[0].astype(jnp.int32)
    dst = edge_index[1].astype(jnp.int32)
    pad_e = EPAD - E
    src_p = jnp.concatenate([src, jnp.zeros((pad_e,), jnp.int32)]).reshape(NW, CH, B)
    dst_p = jnp.concatenate([dst, jnp.full((pad_e,), N, jnp.int32)]).reshape(NW, CH, B)
    x_p = jnp.pad(x, ((0, NPAD - N), (0, 0)))
    batch_p = jnp.concatenate(
        [batch.astype(jnp.int32), jnp.full((NPAD - N,), G, jnp.int32)]
    ).reshape(NGRID, 1, R)

    deg_kernel, edge_kernel = _sc_kernels()
    degp = deg_kernel(dst_p)
    *h1q, dinv = _tc1(x_p, W1, degp.reshape(NC, NPAD, 1))
    s1 = edge_kernel(*h1q, src_p, dst_p)
    h2q = _tc2(s1, *h1q, dinv, b1.reshape(1, D), W2)
    s2 = edge_kernel(*h2q, src_p, dst_p)
    return _tc3(s2, *h2q, dinv, b2.reshape(1, D), batch_p)


# speedup vs baseline: 1.1059x; 1.1059x over previous
import functools

import jax
import jax.numpy as jnp
from jax import lax
from jax.experimental import pallas as pl
from jax.experimental.pallas import tpu as pltpu
from jax.experimental.pallas import tpu_sc as plsc

N = 10000
D = 128
NQ = 4
DQ = D // NQ
G = 64
E = 320000

NPAD = 10240
R = 512
NGRID = NPAD // R
NC = 2
NS = 16
NW = NC * NS
B = 128
CH = 80
EPAD = NW * CH * B
RPT = NPAD // NS
ZR = 128
NBUF = 4
HR = 80
HSEG = NPAD // HR

_sc_params = pltpu.CompilerParams(needs_layout_passes=False,
                                  use_tc_tiling_on_sc=False)


@functools.lru_cache(maxsize=None)
def _sc_kernels():
    mesh = plsc.VectorSubcoreMesh(core_axis_name="c", subcore_axis_name="s",
                                  num_cores=NC, num_subcores=NS)
    deg = functools.partial(
        pl.kernel,
        out_type=jax.ShapeDtypeStruct((NC, HR, HSEG), jnp.float32),
        mesh=mesh,
        compiler_params=_sc_params,
        scratch_types=[
            pltpu.VMEM((CH, B), jnp.int32),
            pltpu.VMEM((HR, HSEG), jnp.float32),
            pltpu.VMEM((HR,), jnp.int32),
            pltpu.VMEM_SHARED((HR, HSEG), jnp.float32),
        ],
    )(_deg_body)
    edge = functools.partial(
        pl.kernel,
        out_type=jax.ShapeDtypeStruct((NC, NQ, NPAD, DQ), jnp.float32),
        mesh=mesh,
        compiler_params=_sc_params,
        scratch_types=[
            pltpu.VMEM((CH, B), jnp.int32),
            pltpu.VMEM((CH, B), jnp.int32),
            [pltpu.VMEM((B, DQ), jnp.float32) for _ in range(NBUF)],
            pltpu.VMEM((ZR, DQ), jnp.float32),
            pltpu.VMEM_SHARED((NPAD, DQ), jnp.float32),
            pltpu.VMEM_SHARED((NPAD, DQ), jnp.float32),
            [pltpu.SemaphoreType.DMA for _ in range(NBUF)],
            [pltpu.SemaphoreType.DMA for _ in range(NBUF)],
        ],
    )(_edge_body)
    return deg, edge


def _deg_body(dst_hbm, out_hbm, idx_v, hist_v, iref, acc_sh):
    c = lax.axis_index("c")
    s = lax.axis_index("s")
    w = s * NC + c
    pltpu.sync_copy(dst_hbm.at[w], idx_v)
    zero16 = jnp.zeros((16,), jnp.float32)

    def _zero(i, carry):
        for k in range(HSEG // 16):
            hist_v[i, pl.ds(k * 16, 16)] = zero16
        return carry

    lax.fori_loop(0, HR, _zero, 0)

    @pl.when(s == 0)
    def _():
        pltpu.sync_copy(hist_v, acc_sh)

    plsc.subcore_barrier()

    one16 = jnp.ones((16,), jnp.float32)

    def _hist(j, carry):
        for k in range(B // 16):
            idx = idx_v[j, pl.ds(k * 16, 16)]
            plsc.addupdate_scatter(hist_v, [idx >> 7, idx & (HSEG - 1)], one16)
        return carry

    lax.fori_loop(0, CH, _hist, 0)

    iota16 = lax.iota(jnp.int32, 16)
    for k in range(HR // 16):
        iref[pl.ds(k * 16, 16)] = iota16 + (k * 16)
    pltpu.sync_copy(hist_v, acc_sh.at[iref], add=True)
    plsc.subcore_barrier()
    rpt = HR // NS
    pltpu.sync_copy(acc_sh.at[pl.ds(rpt * s, rpt)],
                    out_hbm.at[c, pl.ds(rpt * s, rpt)])


def _edge_body(hq0_hbm, hq1_hbm, hq2_hbm, hq3_hbm, src_hbm, dst_hbm, out_hbm,
               sidx, didx, bufs, zbuf, hsh, acc, gsems, ssems):
    c = lax.axis_index("c")
    s = lax.axis_index("s")
    w = s * NC + c
    pltpu.sync_copy(src_hbm.at[w], sidx)
    pltpu.sync_copy(dst_hbm.at[w], didx)

    zero16 = jnp.zeros((16,), jnp.float32)

    def _zb(i, carry):
        for k in range(DQ // 16):
            zbuf[i, pl.ds(k * 16, 16)] = zero16
        return carry

    lax.fori_loop(0, ZR, _zb, 0)

    base = s * RPT

    for p, hq_hbm in enumerate((hq0_hbm, hq1_hbm, hq2_hbm, hq3_hbm)):
        pltpu.sync_copy(hq_hbm.at[pl.ds(base, RPT)], hsh.at[pl.ds(base, RPT)])

        def _za(i, carry):
            pltpu.sync_copy(zbuf, acc.at[pl.ds(base + i * ZR, ZR)])
            return carry

        lax.fori_loop(0, RPT // ZR, _za, 0)
        plsc.subcore_barrier()

        pltpu.async_copy(hsh.at[sidx.at[0]], bufs[0], gsems[0])
        pltpu.async_copy(hsh.at[sidx.at[1]], bufs[1], gsems[1])

        def _body(k, carry):
            v0 = NBUF * k
            for b in range(NBUF):
                v = v0 + b
                pltpu.make_async_copy(hsh.at[sidx.at[v]], bufs[b],
                                      gsems[b]).wait()
                pltpu.async_copy(bufs[b], acc.at[didx.at[v]], ssems[b],
                                 add=True)
                bg = (b + 2) % NBUF

                @pl.when(v >= 2)
                def _():
                    pltpu.make_async_copy(bufs[bg], acc.at[didx.at[v]],
                                          ssems[bg]).wait()

                @pl.when(v + 2 < CH)
                def _():
                    pltpu.async_copy(hsh.at[sidx.at[v + 2]], bufs[bg],
                                     gsems[bg])

            return carry

        lax.fori_loop(0, CH // NBUF, _body, 0)
        for j in (CH - 2, CH - 1):
            pltpu.make_async_copy(bufs[j % NBUF], acc.at[didx.at[j]],
                                  ssems[j % NBUF]).wait()
        plsc.subcore_barrier()
        pltpu.sync_copy(acc.at[pl.ds(base, RPT)],
                        out_hbm.at[c, p, pl.ds(base, RPT)])


def _mm(a, b):
    return lax.dot_general(a, b, (((1,), (0,)), ((), ())),
                           preferred_element_type=jnp.float32,
                           precision=lax.Precision.HIGHEST)


def _tc1_body(x_ref, w_ref, degp_ref, hq0_ref, hq1_ref, hq2_ref, hq3_ref,
              dinv_ref):
    deg = degp_ref[0] + degp_ref[1] + 1.0
    dinv = lax.rsqrt(deg)
    h = dinv * _mm(x_ref[...], w_ref[...])
    for q, ref in enumerate((hq0_ref, hq1_ref, hq2_ref, hq3_ref)):
        ref[...] = h[:, q * DQ:(q + 1) * DQ]
    dinv_ref[...] = dinv


_tc1 = pl.pallas_call(
    _tc1_body,
    grid=(NGRID,),
    in_specs=[
        pl.BlockSpec((R, D), lambda i: (i, 0)),
        pl.BlockSpec((D, D), lambda i: (0, 0)),
        pl.BlockSpec((NC, R, 1), lambda i: (0, i, 0)),
    ],
    out_specs=[pl.BlockSpec((R, DQ), lambda i: (i, 0)) for _ in range(NQ)]
    + [pl.BlockSpec((R, 1), lambda i: (i, 0))],
    out_shape=[jax.ShapeDtypeStruct((NPAD, DQ), jnp.float32)
               for _ in range(NQ)]
    + [jax.ShapeDtypeStruct((NPAD, 1), jnp.float32)],
)


def _node_emb(s_ref, hq_refs, dinv_ref, b_ref):
    dinv = dinv_ref[...]
    quarters = [dinv * (s_ref[0, q] + s_ref[1, q] + hq_refs[q][...])
                for q in range(NQ)]
    return jnp.concatenate(quarters, axis=1) + b_ref[...]


def _tc2_body(s_ref, hq0_ref, hq1_ref, hq2_ref, hq3_ref, dinv_ref, b1_ref,
              w2_ref, oq0_ref, oq1_ref, oq2_ref, oq3_ref):
    h1 = jnp.maximum(
        _node_emb(s_ref, (hq0_ref, hq1_ref, hq2_ref, hq3_ref), dinv_ref,
                  b1_ref), 0.0)
    h2 = dinv_ref[...] * _mm(h1, w2_ref[...])
    for q, ref in enumerate((oq0_ref, oq1_ref, oq2_ref, oq3_ref)):
        ref[...] = h2[:, q * DQ:(q + 1) * DQ]


_tc2 = pl.pallas_call(
    _tc2_body,
    grid=(NGRID,),
    in_specs=[
        pl.BlockSpec((NC, NQ, R, DQ), lambda i: (0, 0, i, 0)),
    ] + [pl.BlockSpec((R, DQ), lambda i: (i, 0)) for _ in range(NQ)] + [
        pl.BlockSpec((R, 1), lambda i: (i, 0)),
        pl.BlockSpec((1, D), lambda i: (0, 0)),
        pl.BlockSpec((D, D), lambda i: (0, 0)),
    ],
    out_specs=[pl.BlockSpec((R, DQ), lambda i: (i, 0)) for _ in range(NQ)],
    out_shape=[jax.ShapeDtypeStruct((NPAD, DQ), jnp.float32)
               for _ in range(NQ)],
)


def _tc3_body(s_ref, hq0_ref, hq1_ref, hq2_ref, hq3_ref, dinv_ref, b2_ref,
              batch_ref, out_ref, acc_ref, cnt_ref):
    i = pl.program_id(0)
    node = _node_emb(s_ref, (hq0_ref, hq1_ref, hq2_ref, hq3_ref), dinv_ref,
                     b2_ref)
    batch_blk = batch_ref[0]
    iota_g = lax.broadcasted_iota(jnp.int32, (G, R), 0)
    mask = (batch_blk == iota_g).astype(jnp.float32)
    contrib = _mm(mask, node)
    cntc = jnp.sum(mask, axis=1, keepdims=True)

    @pl.when(i == 0)
    def _():
        acc_ref[...] = contrib
        cnt_ref[...] = cntc

    @pl.when(i > 0)
    def _():
        acc_ref[...] += contrib
        cnt_ref[...] += cntc

    @pl.when(i == NGRID - 1)
    def _():
        out_ref[...] = acc_ref[...] / jnp.maximum(cnt_ref[...], 1.0)


_tc3 = pl.pallas_call(
    _tc3_body,
    grid=(NGRID,),
    in_specs=[
        pl.BlockSpec((NC, NQ, R, DQ), lambda i: (0, 0, i, 0)),
    ] + [pl.BlockSpec((R, DQ), lambda i: (i, 0)) for _ in range(NQ)] + [
        pl.BlockSpec((R, 1), lambda i: (i, 0)),
        pl.BlockSpec((1, D), lambda i: (0, 0)),
        pl.BlockSpec((1, 1, R), lambda i: (i, 0, 0)),
    ],
    out_specs=pl.BlockSpec((G, D), lambda i: (0, 0)),
    out_shape=jax.ShapeDtypeStruct((G, D), jnp.float32),
    scratch_shapes=[
        pltpu.VMEM((G, D), jnp.float32),
        pltpu.VMEM((G, 1), jnp.float32),
    ],
)


def kernel(x, edge_index, batch, W1, b1, W2, b2):
    src = edge_index[0].astype(jnp.int32)
    dst = edge_index[1].astype(jnp.int32)
    pad_e = EPAD - E
    src_p = jnp.concatenate([src, jnp.zeros((pad_e,), jnp.int32)]).reshape(NW, CH, B)
    dst_p = jnp.concatenate([dst, jnp.full((pad_e,), N, jnp.int32)]).reshape(NW, CH, B)
    x_p = jnp.pad(x, ((0, NPAD - N), (0, 0)))
    batch_p = jnp.concatenate(
        [batch.astype(jnp.int32), jnp.full((NPAD - N,), G, jnp.int32)]
    ).reshape(NGRID, 1, R)

    deg_kernel, edge_kernel = _sc_kernels()
    degp = deg_kernel(dst_p)
    *h1q, dinv = _tc1(x_p, W1, degp.reshape(NC, NPAD, 1))
    s1 = edge_kernel(*h1q, src_p, dst_p)
    h2q = _tc2(s1, *h1q, dinv, b1.reshape(1, D), W2)
    s2 = edge_kernel(*h2q, src_p, dst_p)
    return _tc3(s2, *h2q, dinv, b2.reshape(1, D), batch_p)


# speedup vs baseline: 1.1131x; 1.0065x over previous
import functools

import jax
import jax.numpy as jnp
from jax import lax
from jax.experimental import pallas as pl
from jax.experimental.pallas import tpu as pltpu
from jax.experimental.pallas import tpu_sc as plsc

N = 10000
D = 128
NQ = 4
DQ = D // NQ
G = 64
E = 320000

NPAD = 10240
R = 512
NGRID = NPAD // R
NC = 2
NS = 16
NW = NC * NS
B = 128
CH = 80
EPAD = NW * CH * B
RPT = NPAD // NS
ZR = 128
NBUF = 4
HR = 80
HSEG = NPAD // HR

_sc_params = pltpu.CompilerParams(needs_layout_passes=False,
                                  use_tc_tiling_on_sc=False)


@functools.lru_cache(maxsize=None)
def _sc_kernels():
    mesh = plsc.VectorSubcoreMesh(core_axis_name="c", subcore_axis_name="s",
                                  num_cores=NC, num_subcores=NS)
    deg = functools.partial(
        pl.kernel,
        out_type=jax.ShapeDtypeStruct((NC, HR, HSEG), jnp.float32),
        mesh=mesh,
        compiler_params=_sc_params,
        scratch_types=[
            pltpu.VMEM((CH, B), jnp.int32),
            pltpu.VMEM((HR, HSEG), jnp.float32),
            pltpu.VMEM((HR,), jnp.int32),
            pltpu.VMEM_SHARED((HR, HSEG), jnp.float32),
        ],
    )(_deg_body)
    edge = functools.partial(
        pl.kernel,
        out_type=jax.ShapeDtypeStruct((NC, NQ, NPAD, DQ), jnp.float32),
        mesh=mesh,
        compiler_params=_sc_params,
        scratch_types=[
            pltpu.VMEM((CH, B), jnp.int32),
            pltpu.VMEM((CH, B), jnp.int32),
            [pltpu.VMEM((B, DQ), jnp.float32) for _ in range(NBUF)],
            pltpu.VMEM((RPT, DQ), jnp.float32),
            pltpu.VMEM_SHARED((NPAD, DQ), jnp.float32),
            pltpu.VMEM_SHARED((NPAD, DQ), jnp.float32),
            [pltpu.SemaphoreType.DMA for _ in range(NBUF)],
            [pltpu.SemaphoreType.DMA for _ in range(NBUF)],
        ],
    )(_edge_body)
    return deg, edge


def _deg_body(dst_hbm, out_hbm, idx_v, hist_v, iref, acc_sh):
    c = lax.axis_index("c")
    s = lax.axis_index("s")
    w = s * NC + c
    pltpu.sync_copy(dst_hbm.at[w], idx_v)
    zero16 = jnp.zeros((16,), jnp.float32)

    def _zero(i, carry):
        for k in range(HSEG // 16):
            hist_v[i, pl.ds(k * 16, 16)] = zero16
        return carry

    lax.fori_loop(0, HR, _zero, 0)

    @pl.when(s == 0)
    def _():
        pltpu.sync_copy(hist_v, acc_sh)

    plsc.subcore_barrier()

    one16 = jnp.ones((16,), jnp.float32)

    def _hist(j, carry):
        for k in range(B // 16):
            idx = idx_v[j, pl.ds(k * 16, 16)]
            plsc.addupdate_scatter(hist_v, [idx >> 7, idx & (HSEG - 1)], one16)
        return carry

    lax.fori_loop(0, CH, _hist, 0)

    iota16 = lax.iota(jnp.int32, 16)
    for k in range(HR // 16):
        iref[pl.ds(k * 16, 16)] = iota16 + (k * 16)
    pltpu.sync_copy(hist_v, acc_sh.at[iref], add=True)
    plsc.subcore_barrier()
    rpt = HR // NS
    pltpu.sync_copy(acc_sh.at[pl.ds(rpt * s, rpt)],
                    out_hbm.at[c, pl.ds(rpt * s, rpt)])


def _edge_body(hq0_hbm, hq1_hbm, hq2_hbm, hq3_hbm, src_hbm, dst_hbm, out_hbm,
               sidx, didx, bufs, zbuf, hsh, acc, gsems, ssems):
    c = lax.axis_index("c")
    s = lax.axis_index("s")
    w = s * NC + c
    pltpu.sync_copy(src_hbm.at[w], sidx)
    pltpu.sync_copy(dst_hbm.at[w], didx)

    zero16 = jnp.zeros((16,), jnp.float32)

    def _zb(i, carry):
        for k in range(DQ // 16):
            zbuf[i, pl.ds(k * 16, 16)] = zero16
        return carry

    lax.fori_loop(0, RPT, _zb, 0)

    base = s * RPT

    for p, hq_hbm in enumerate((hq0_hbm, hq1_hbm, hq2_hbm, hq3_hbm)):
        pltpu.async_copy(hq_hbm.at[pl.ds(base, RPT)],
                         hsh.at[pl.ds(base, RPT)], gsems[0])
        pltpu.async_copy(zbuf, acc.at[pl.ds(base, RPT)], gsems[1])
        pltpu.make_async_copy(hq_hbm.at[pl.ds(base, RPT)],
                              hsh.at[pl.ds(base, RPT)], gsems[0]).wait()
        pltpu.make_async_copy(zbuf, acc.at[pl.ds(base, RPT)],
                              gsems[1]).wait()
        plsc.subcore_barrier()

        pltpu.async_copy(hsh.at[sidx.at[0]], bufs[0], gsems[0])
        pltpu.async_copy(hsh.at[sidx.at[1]], bufs[1], gsems[1])

        def _body(k, carry):
            v0 = NBUF * k
            for b in range(NBUF):
                v = v0 + b
                pltpu.make_async_copy(hsh.at[sidx.at[v]], bufs[b],
                                      gsems[b]).wait()
                pltpu.async_copy(bufs[b], acc.at[didx.at[v]], ssems[b],
                                 add=True)
                bg = (b + 2) % NBUF

                @pl.when(v >= 2)
                def _():
                    pltpu.make_async_copy(bufs[bg], acc.at[didx.at[v]],
                                          ssems[bg]).wait()

                @pl.when(v + 2 < CH)
                def _():
                    pltpu.async_copy(hsh.at[sidx.at[v + 2]], bufs[bg],
                                     gsems[bg])

            return carry

        lax.fori_loop(0, CH // NBUF, _body, 0)
        for j in (CH - 2, CH - 1):
            pltpu.make_async_copy(bufs[j % NBUF], acc.at[didx.at[j]],
                                  ssems[j % NBUF]).wait()
        plsc.subcore_barrier()
        pltpu.sync_copy(acc.at[pl.ds(base, RPT)],
                        out_hbm.at[c, p, pl.ds(base, RPT)])


def _mm(a, b):
    return lax.dot_general(a, b, (((1,), (0,)), ((), ())),
                           preferred_element_type=jnp.float32,
                           precision=lax.Precision.HIGHEST)


def _tc1_body(x_ref, w_ref, degp_ref, hq0_ref, hq1_ref, hq2_ref, hq3_ref,
              dinv_ref):
    deg = degp_ref[0] + degp_ref[1] + 1.0
    dinv = lax.rsqrt(deg)
    h = dinv * _mm(x_ref[...], w_ref[...])
    for q, ref in enumerate((hq0_ref, hq1_ref, hq2_ref, hq3_ref)):
        ref[...] = h[:, q * DQ:(q + 1) * DQ]
    dinv_ref[...] = dinv


_tc1 = pl.pallas_call(
    _tc1_body,
    grid=(NGRID,),
    in_specs=[
        pl.BlockSpec((R, D), lambda i: (i, 0)),
        pl.BlockSpec((D, D), lambda i: (0, 0)),
        pl.BlockSpec((NC, R, 1), lambda i: (0, i, 0)),
    ],
    out_specs=[pl.BlockSpec((R, DQ), lambda i: (i, 0)) for _ in range(NQ)]
    + [pl.BlockSpec((R, 1), lambda i: (i, 0))],
    out_shape=[jax.ShapeDtypeStruct((NPAD, DQ), jnp.float32)
               for _ in range(NQ)]
    + [jax.ShapeDtypeStruct((NPAD, 1), jnp.float32)],
)


def _node_emb(s_ref, hq_refs, dinv_ref, b_ref):
    dinv = dinv_ref[...]
    quarters = [dinv * (s_ref[0, q] + s_ref[1, q] + hq_refs[q][...])
                for q in range(NQ)]
    return jnp.concatenate(quarters, axis=1) + b_ref[...]


def _tc2_body(s_ref, hq0_ref, hq1_ref, hq2_ref, hq3_ref, dinv_ref, b1_ref,
              w2_ref, oq0_ref, oq1_ref, oq2_ref, oq3_ref):
    h1 = jnp.maximum(
        _node_emb(s_ref, (hq0_ref, hq1_ref, hq2_ref, hq3_ref), dinv_ref,
                  b1_ref), 0.0)
    h2 = dinv_ref[...] * _mm(h1, w2_ref[...])
    for q, ref in enumerate((oq0_ref, oq1_ref, oq2_ref, oq3_ref)):
        ref[...] = h2[:, q * DQ:(q + 1) * DQ]


_tc2 = pl.pallas_call(
    _tc2_body,
    grid=(NGRID,),
    in_specs=[
        pl.BlockSpec((NC, NQ, R, DQ), lambda i: (0, 0, i, 0)),
    ] + [pl.BlockSpec((R, DQ), lambda i: (i, 0)) for _ in range(NQ)] + [
        pl.BlockSpec((R, 1), lambda i: (i, 0)),
        pl.BlockSpec((1, D), lambda i: (0, 0)),
        pl.BlockSpec((D, D), lambda i: (0, 0)),
    ],
    out_specs=[pl.BlockSpec((R, DQ), lambda i: (i, 0)) for _ in range(NQ)],
    out_shape=[jax.ShapeDtypeStruct((NPAD, DQ), jnp.float32)
               for _ in range(NQ)],
)


def _tc3_body(s_ref, hq0_ref, hq1_ref, hq2_ref, hq3_ref, dinv_ref, b2_ref,
              batch_ref, out_ref, acc_ref, cnt_ref):
    i = pl.program_id(0)
    node = _node_emb(s_ref, (hq0_ref, hq1_ref, hq2_ref, hq3_ref), dinv_ref,
                     b2_ref)
    batch_blk = batch_ref[0]
    iota_g = lax.broadcasted_iota(jnp.int32, (G, R), 0)
    mask = (batch_blk == iota_g).astype(jnp.float32)
    contrib = _mm(mask, node)
    cntc = jnp.sum(mask, axis=1, keepdims=True)

    @pl.when(i == 0)
    def _():
        acc_ref[...] = contrib
        cnt_ref[...] = cntc

    @pl.when(i > 0)
    def _():
        acc_ref[...] += contrib
        cnt_ref[...] += cntc

    @pl.when(i == NGRID - 1)
    def _():
        out_ref[...] = acc_ref[...] / jnp.maximum(cnt_ref[...], 1.0)


_tc3 = pl.pallas_call(
    _tc3_body,
    grid=(NGRID,),
    in_specs=[
        pl.BlockSpec((NC, NQ, R, DQ), lambda i: (0, 0, i, 0)),
    ] + [pl.BlockSpec((R, DQ), lambda i: (i, 0)) for _ in range(NQ)] + [
        pl.BlockSpec((R, 1), lambda i: (i, 0)),
        pl.BlockSpec((1, D), lambda i: (0, 0)),
        pl.BlockSpec((1, 1, R), lambda i: (i, 0, 0)),
    ],
    out_specs=pl.BlockSpec((G, D), lambda i: (0, 0)),
    out_shape=jax.ShapeDtypeStruct((G, D), jnp.float32),
    scratch_shapes=[
        pltpu.VMEM((G, D), jnp.float32),
        pltpu.VMEM((G, 1), jnp.float32),
    ],
)


def kernel(x, edge_index, batch, W1, b1, W2, b2):
    src = edge_index[0].astype(jnp.int32)
    dst = edge_index[1].astype(jnp.int32)
    pad_e = EPAD - E
    src_p = jnp.concatenate([src, jnp.zeros((pad_e,), jnp.int32)]).reshape(NW, CH, B)
    dst_p = jnp.concatenate([dst, jnp.full((pad_e,), N, jnp.int32)]).reshape(NW, CH, B)
    x_p = jnp.pad(x, ((0, NPAD - N), (0, 0)))
    batch_p = jnp.concatenate(
        [batch.astype(jnp.int32), jnp.full((NPAD - N,), G, jnp.int32)]
    ).reshape(NGRID, 1, R)

    deg_kernel, edge_kernel = _sc_kernels()
    degp = deg_kernel(dst_p)
    *h1q, dinv = _tc1(x_p, W1, degp.reshape(NC, NPAD, 1))
    s1 = edge_kernel(*h1q, src_p, dst_p)
    h2q = _tc2(s1, *h1q, dinv, b1.reshape(1, D), W2)
    s2 = edge_kernel(*h2q, src_p, dst_p)
    return _tc3(s2, *h2q, dinv, b2.reshape(1, D), batch_p)


# speedup vs baseline: 1.1131x; 1.0000x over previous
import functools

import jax
import jax.numpy as jnp
from jax import lax
from jax.experimental import pallas as pl
from jax.experimental.pallas import tpu as pltpu
from jax.experimental.pallas import tpu_sc as plsc

N = 10000
D = 128
NQ = 4
DQ = D // NQ
G = 64
E = 320000

NPAD = 10240
R = 512
NGRID = NPAD // R
NC = 2
NS = 16
NW = NC * NS
B = 128
CH = 80
EPAD = NW * CH * B
RPT = NPAD // NS
NBUF = 4
HR = 80
HSEG = NPAD // HR

_sc_params = pltpu.CompilerParams(needs_layout_passes=False,
                                  use_tc_tiling_on_sc=False)


@functools.lru_cache(maxsize=None)
def _sc_kernels():
    mesh = plsc.VectorSubcoreMesh(core_axis_name="c", subcore_axis_name="s",
                                  num_cores=NC, num_subcores=NS)
    deg = functools.partial(
        pl.kernel,
        out_type=jax.ShapeDtypeStruct((NC, HR, HSEG), jnp.float32),
        mesh=mesh,
        compiler_params=_sc_params,
        scratch_types=[
            pltpu.VMEM((CH, B), jnp.int32),
            pltpu.VMEM((HR, HSEG), jnp.float32),
            pltpu.VMEM((HR,), jnp.int32),
            pltpu.VMEM_SHARED((HR, HSEG), jnp.float32),
        ],
    )(_deg_body)
    edge = functools.partial(
        pl.kernel,
        out_type=jax.ShapeDtypeStruct((NC, NQ, NPAD, DQ), jnp.float32),
        mesh=mesh,
        compiler_params=_sc_params,
        scratch_types=[
            pltpu.VMEM((CH, B), jnp.int32),
            pltpu.VMEM((CH, B), jnp.int32),
            [pltpu.VMEM((B, DQ), jnp.float32) for _ in range(NBUF)],
            pltpu.VMEM((RPT, DQ), jnp.float32),
            pltpu.VMEM_SHARED((NPAD, DQ), jnp.float32),
            pltpu.VMEM_SHARED((NPAD, DQ), jnp.float32),
            [pltpu.SemaphoreType.DMA for _ in range(NBUF)],
            [pltpu.SemaphoreType.DMA for _ in range(NBUF)],
        ],
    )(_edge_body)
    return deg, edge


def _deg_body(dst_hbm, out_hbm, idx_v, hist_v, iref, acc_sh):
    c = lax.axis_index("c")
    s = lax.axis_index("s")
    w = s * NC + c
    pltpu.sync_copy(dst_hbm.at[w], idx_v)
    zero16 = jnp.zeros((16,), jnp.float32)

    def _zero(i, carry):
        for k in range(HSEG // 16):
            hist_v[i, pl.ds(k * 16, 16)] = zero16
        return carry

    lax.fori_loop(0, HR, _zero, 0)

    @pl.when(s == 0)
    def _():
        pltpu.sync_copy(hist_v, acc_sh)

    plsc.subcore_barrier()

    one16 = jnp.ones((16,), jnp.float32)

    def _hist(j, carry):
        for k in range(B // 16):
            idx = idx_v[j, pl.ds(k * 16, 16)]
            plsc.addupdate_scatter(hist_v, [idx >> 7, idx & (HSEG - 1)], one16)
        return carry

    lax.fori_loop(0, CH, _hist, 0)

    iota16 = lax.iota(jnp.int32, 16)
    for k in range(HR // 16):
        iref[pl.ds(k * 16, 16)] = iota16 + (k * 16)
    pltpu.sync_copy(hist_v, acc_sh.at[iref], add=True)
    plsc.subcore_barrier()
    rpt = HR // NS
    pltpu.sync_copy(acc_sh.at[pl.ds(rpt * s, rpt)],
                    out_hbm.at[c, pl.ds(rpt * s, rpt)])


def _edge_body(hq0_hbm, hq1_hbm, hq2_hbm, hq3_hbm, src_hbm, dst_hbm, out_hbm,
               sidx, didx, bufs, zbuf, hsh, acc, gsems, ssems):
    c = lax.axis_index("c")
    s = lax.axis_index("s")
    w = s * NC + c
    pltpu.sync_copy(src_hbm.at[w], sidx)
    pltpu.sync_copy(dst_hbm.at[w], didx)

    zero16 = jnp.zeros((16,), jnp.float32)

    def _zb(i, carry):
        for k in range(DQ // 16):
            zbuf[i, pl.ds(k * 16, 16)] = zero16
        return carry

    lax.fori_loop(0, RPT, _zb, 0)

    base = s * RPT

    for p, hq_hbm in enumerate((hq0_hbm, hq1_hbm, hq2_hbm, hq3_hbm)):
        pltpu.async_copy(hq_hbm.at[pl.ds(base, RPT)],
                         hsh.at[pl.ds(base, RPT)], gsems[0])
        pltpu.async_copy(zbuf, acc.at[pl.ds(base, RPT)], gsems[1])
        pltpu.make_async_copy(hq_hbm.at[pl.ds(base, RPT)],
                              hsh.at[pl.ds(base, RPT)], gsems[0]).wait()
        pltpu.make_async_copy(zbuf, acc.at[pl.ds(base, RPT)],
                              gsems[1]).wait()
        plsc.subcore_barrier()

        pltpu.async_copy(hsh.at[sidx.at[0]], bufs[0], gsems[0])
        pltpu.async_copy(hsh.at[sidx.at[1]], bufs[1], gsems[1])

        def _body(k, carry):
            v0 = NBUF * k
            for b in range(NBUF):
                v = v0 + b
                pltpu.make_async_copy(hsh.at[sidx.at[v]], bufs[b],
                                      gsems[b]).wait()
                pltpu.async_copy(bufs[b], acc.at[didx.at[v]], ssems[b],
                                 add=True)
                bg = (b + 2) % NBUF

                @pl.when(v >= 2)
                def _():
                    pltpu.make_async_copy(bufs[bg], acc.at[didx.at[v]],
                                          ssems[bg]).wait()

                @pl.when(v + 2 < CH)
                def _():
                    pltpu.async_copy(hsh.at[sidx.at[v + 2]], bufs[bg],
                                     gsems[bg])

            return carry

        lax.fori_loop(0, CH // NBUF, _body, 0)
        for j in (CH - 2, CH - 1):
            pltpu.make_async_copy(bufs[j % NBUF], acc.at[didx.at[j]],
                                  ssems[j % NBUF]).wait()
        plsc.subcore_barrier()
        pltpu.sync_copy(acc.at[pl.ds(base, RPT)],
                        out_hbm.at[c, p, pl.ds(base, RPT)])


def _mm(a, b):
    return lax.dot_general(a, b, (((1,), (0,)), ((), ())),
                           preferred_element_type=jnp.float32,
                           precision=lax.Precision.HIGHEST)


def _tc1_body(x_ref, w_ref, degp_ref, hq0_ref, hq1_ref, hq2_ref, hq3_ref,
              dinv_ref):
    deg = degp_ref[0] + degp_ref[1] + 1.0
    dinv = lax.rsqrt(deg)
    h = dinv * _mm(x_ref[...], w_ref[...])
    for q, ref in enumerate((hq0_ref, hq1_ref, hq2_ref, hq3_ref)):
        ref[...] = h[:, q * DQ:(q + 1) * DQ]
    dinv_ref[...] = dinv


_tc1 = pl.pallas_call(
    _tc1_body,
    grid=(NGRID,),
    in_specs=[
        pl.BlockSpec((R, D), lambda i: (i, 0)),
        pl.BlockSpec((D, D), lambda i: (0, 0)),
        pl.BlockSpec((NC, R, 1), lambda i: (0, i, 0)),
    ],
    out_specs=[pl.BlockSpec((R, DQ), lambda i: (i, 0)) for _ in range(NQ)]
    + [pl.BlockSpec((R, 1), lambda i: (i, 0))],
    out_shape=[jax.ShapeDtypeStruct((NPAD, DQ), jnp.float32)
               for _ in range(NQ)]
    + [jax.ShapeDtypeStruct((NPAD, 1), jnp.float32)],
)


def _node_emb(s_ref, hq_refs, dinv_ref, b_ref):
    dinv = dinv_ref[...]
    quarters = [dinv * (s_ref[0, q] + s_ref[1, q] + hq_refs[q][...])
                for q in range(NQ)]
    return jnp.concatenate(quarters, axis=1) + b_ref[...]


def _tc2_body(s_ref, hq0_ref, hq1_ref, hq2_ref, hq3_ref, dinv_ref, b1_ref,
              w2_ref, oq0_ref, oq1_ref, oq2_ref, oq3_ref):
    h1 = jnp.maximum(
        _node_emb(s_ref, (hq0_ref, hq1_ref, hq2_ref, hq3_ref), dinv_ref,
                  b1_ref), 0.0)
    h2 = dinv_ref[...] * _mm(h1, w2_ref[...])
    for q, ref in enumerate((oq0_ref, oq1_ref, oq2_ref, oq3_ref)):
        ref[...] = h2[:, q * DQ:(q + 1) * DQ]


_tc2 = pl.pallas_call(
    _tc2_body,
    grid=(NGRID,),
    in_specs=[
        pl.BlockSpec((NC, NQ, R, DQ), lambda i: (0, 0, i, 0)),
    ] + [pl.BlockSpec((R, DQ), lambda i: (i, 0)) for _ in range(NQ)] + [
        pl.BlockSpec((R, 1), lambda i: (i, 0)),
        pl.BlockSpec((1, D), lambda i: (0, 0)),
        pl.BlockSpec((D, D), lambda i: (0, 0)),
    ],
    out_specs=[pl.BlockSpec((R, DQ), lambda i: (i, 0)) for _ in range(NQ)],
    out_shape=[jax.ShapeDtypeStruct((NPAD, DQ), jnp.float32)
               for _ in range(NQ)],
)


def _tc3_body(s_ref, hq0_ref, hq1_ref, hq2_ref, hq3_ref, dinv_ref, b2_ref,
              batch_ref, out_ref, acc_ref, cnt_ref):
    i = pl.program_id(0)
    node = _node_emb(s_ref, (hq0_ref, hq1_ref, hq2_ref, hq3_ref), dinv_ref,
                     b2_ref)
    batch_blk = batch_ref[0]
    iota_g = lax.broadcasted_iota(jnp.int32, (G, R), 0)
    mask = (batch_blk == iota_g).astype(jnp.float32)
    contrib = _mm(mask, node)
    cntc = jnp.sum(mask, axis=1, keepdims=True)

    @pl.when(i == 0)
    def _():
        acc_ref[...] = contrib
        cnt_ref[...] = cntc

    @pl.when(i > 0)
    def _():
        acc_ref[...] += contrib
        cnt_ref[...] += cntc

    @pl.when(i == NGRID - 1)
    def _():
        out_ref[...] = acc_ref[...] / jnp.maximum(cnt_ref[...], 1.0)


_tc3 = pl.pallas_call(
    _tc3_body,
    grid=(NGRID,),
    in_specs=[
        pl.BlockSpec((NC, NQ, R, DQ), lambda i: (0, 0, i, 0)),
    ] + [pl.BlockSpec((R, DQ), lambda i: (i, 0)) for _ in range(NQ)] + [
        pl.BlockSpec((R, 1), lambda i: (i, 0)),
        pl.BlockSpec((1, D), lambda i: (0, 0)),
        pl.BlockSpec((1, 1, R), lambda i: (i, 0, 0)),
    ],
    out_specs=pl.BlockSpec((G, D), lambda i: (0, 0)),
    out_shape=jax.ShapeDtypeStruct((G, D), jnp.float32),
    scratch_shapes=[
        pltpu.VMEM((G, D), jnp.float32),
        pltpu.VMEM((G, 1), jnp.float32),
    ],
)


def kernel(x, edge_index, batch, W1, b1, W2, b2):
    src = edge_index[0].astype(jnp.int32)
    dst = edge_index[1].astype(jnp.int32)
    pad_e = EPAD - E
    src_p = jnp.concatenate([src, jnp.zeros((pad_e,), jnp.int32)]).reshape(NW, CH, B)
    dst_p = jnp.concatenate([dst, jnp.full((pad_e,), N, jnp.int32)]).reshape(NW, CH, B)
    x_p = jnp.pad(x, ((0, NPAD - N), (0, 0)))
    batch_p = jnp.concatenate(
        [batch.astype(jnp.int32), jnp.full((NPAD - N,), G, jnp.int32)]
    ).reshape(NGRID, 1, R)

    deg_kernel, edge_kernel = _sc_kernels()
    degp = deg_kernel(dst_p)
    *h1q, dinv = _tc1(x_p, W1, degp.reshape(NC, NPAD, 1))
    s1 = edge_kernel(*h1q, src_p, dst_p)
    h2q = _tc2(s1, *h1q, dinv, b1.reshape(1, D), W2)
    s2 = edge_kernel(*h2q, src_p, dst_p)
    return _tc3(s2, *h2q, dinv, b2.reshape(1, D), batch_p)
